# baseline (speedup 1.0000x reference)
import functools

import jax
import jax.numpy as jnp
from jax import lax
from jax.experimental import pallas as pl
from jax.experimental.pallas import tpu as pltpu
from jax.experimental.pallas import tpu_sc as plsc

NBINS = 256
ROWS = 128
N = 64 * 64 * 64
CENTER = N // 2
NORM = 65 * 65 * 65
LOG2E = 1.4426950408889634

NC, NS = 2, 16
NW = NC * NS
RPW = ROWS // NW
CH = 16384
CPR = N // CH
TCH = RPW * CPR
NBUF = 4
OUTW = 32

_mesh = plsc.VectorSubcoreMesh(core_axis_name="c", subcore_axis_name="s")


@functools.partial(
    pl.kernel,
    mesh=_mesh,
    out_type=jax.ShapeDtypeStruct((ROWS * OUTW,), jnp.float32),
    scratch_types=[
        pltpu.VMEM((CH,), jnp.float32),
        pltpu.VMEM((CH,), jnp.float32),
        pltpu.VMEM((CH,), jnp.float32),
        pltpu.VMEM((CH,), jnp.float32),
        pltpu.VMEM((RPW * 16,), jnp.float32),
        pltpu.VMEM((16,), jnp.float32),
        pltpu.SemaphoreType.DMA,
        pltpu.SemaphoreType.DMA,
        pltpu.SemaphoreType.DMA,
        pltpu.SemaphoreType.DMA,
    ],
)
def _sc_rowsum(x_hbm, out_hbm, b0, b1, b2, b3, accv, cenb, s0, s1, s2, s3):
    w = lax.axis_index("c") * NS + lax.axis_index("s")
    base = w * (RPW * N)
    bufs = (b0, b1, b2, b3)
    sems = (s0, s1, s2, s3)

    for k in range(RPW):
        accv[pl.ds(k * 16, 16)] = jnp.zeros((16,), jnp.float32)

    for t in range(NBUF - 1):
        pltpu.async_copy(x_hbm.at[pl.ds(base + t * CH, CH)], bufs[t], sems[t])

    for t in range(TCH):
        nxt = t + NBUF - 1
        if nxt < TCH:
            pltpu.async_copy(
                x_hbm.at[pl.ds(base + nxt * CH, CH)],
                bufs[nxt % NBUF],
                sems[nxt % NBUF],
            )
        pltpu.make_async_copy(
            x_hbm.at[pl.ds(base + t * CH, CH)], bufs[t % NBUF], sems[t % NBUF]
        ).wait()
        buf = bufs[t % NBUF]
        k = t // CPR

        @pl.loop(0, CH, step=128)
        def _(c0, _buf=buf, _k=k):
            v = _buf[pl.ds(c0, 16)]
            for j in range(1, 8):
                v = v + _buf[pl.ds(c0 + 16 * j, 16)]
            accv[pl.ds(_k * 16, 16)] += v

    for k in range(RPW):
        off = base * 0 + (w * RPW + k) * OUTW
        pltpu.sync_copy(accv.at[pl.ds(k * 16, 16)], out_hbm.at[pl.ds(off, 16)])
        pltpu.sync_copy(x_hbm.at[pl.ds(base + k * N + CENTER, 16)], cenb)
        pltpu.sync_copy(cenb, out_hbm.at[pl.ds(off + 16, 16)])


def _finish_body(p_ref, out_ref):
    part = p_ref[...]
    sums = part[:, 0:16].sum(axis=1, keepdims=True)
    cen = part[:, 16:17]
    nb = (sums - cen) * (1.0 / (N - 1))
    vals = jnp.concatenate([cen, nb], axis=0)
    idx = jnp.floor(vals * NBINS).astype(jnp.int32)
    valid = (vals >= 0.0) & (vals <= 1.0)
    idx = jnp.minimum(idx, NBINS - 1)
    bins = lax.broadcasted_iota(jnp.int32, (2 * ROWS, NBINS), 1)
    match = (idx == bins) & valid
    counts = jnp.sum(match.astype(jnp.float32), axis=0, keepdims=True)
    p = counts * (1.0 / NORM)
    e = -jnp.sum(p * (jnp.log(p + 1e-10) * LOG2E), axis=1, keepdims=True)
    out_ref[...] = e


def kernel(F):
    x = F.reshape(-1)
    part = _sc_rowsum(x).reshape(ROWS, OUTW)
    out = pl.pallas_call(
        _finish_body,
        grid=(1,),
        in_specs=[pl.BlockSpec((ROWS, OUTW), lambda i: (0, 0))],
        out_specs=pl.BlockSpec((1, 1), lambda i: (0, 0)),
        out_shape=jax.ShapeDtypeStruct((1, 1), jnp.float32),
    )(part)
    return out.reshape(())


# speedup vs baseline: 2.8387x; 2.8387x over previous
import jax
import jax.numpy as jnp
from jax import lax
from jax.experimental import pallas as pl
from jax.experimental.pallas import tpu as pltpu

NBINS = 256
B, C, H, W, D = 4, 32, 64, 64, 64
ROWS = B * C
N = H * W * D
CENTER_H = (N // 2) // (W * D)
NORM = 65 * 65 * 65
LOG2E = 1.4426950408889634


def _entropy_body(x_ref, out_ref, acc_ref, cen_ref):
    i = pl.program_id(0)

    blk = x_ref[0, 0]
    s = blk.sum(axis=0).sum(axis=0, keepdims=True)
    acc_ref[pl.ds(i, 1), :] = s
    cen_ref[pl.ds(i, 1), :] = blk[CENTER_H, 0:1, 0:1]

    @pl.when(i == ROWS - 1)
    def _():
        sums = acc_ref[...].sum(axis=1, keepdims=True)
        cen = cen_ref[...]
        nb = (sums - cen) * (1.0 / (N - 1))
        vals = jnp.concatenate([cen, nb], axis=0)
        idx = jnp.floor(vals * NBINS).astype(jnp.int32)
        valid = (vals >= 0.0) & (vals <= 1.0)
        idx = jnp.minimum(idx, NBINS - 1)
        bins = lax.broadcasted_iota(jnp.int32, (2 * ROWS, NBINS), 1)
        match = (idx == bins) & valid
        counts = jnp.sum(match.astype(jnp.float32), axis=0, keepdims=True)
        p = counts * (1.0 / NORM)
        e = -jnp.sum(p * (jnp.log(p + 1e-10) * LOG2E), axis=1, keepdims=True)
        out_ref[...] = e


def kernel(F):
    out = pl.pallas_call(
        _entropy_body,
        grid=(ROWS,),
        in_specs=[
            pl.BlockSpec((1, 1, H, W, D), lambda i: (i // C, i % C, 0, 0, 0))
        ],
        out_specs=pl.BlockSpec((1, 1), lambda i: (0, 0)),
        out_shape=jax.ShapeDtypeStruct((1, 1), jnp.float32),
        scratch_shapes=[
            pltpu.VMEM((ROWS, D), jnp.float32),
            pltpu.VMEM((ROWS, 1), jnp.float32),
        ],
    )(F)
    return out.reshape(())


# speedup vs baseline: 3.8722x; 1.3641x over previous
import jax
import jax.numpy as jnp
from jax import lax
from jax.experimental import pallas as pl
from jax.experimental.pallas import tpu as pltpu

NBINS = 256
B, C, H, W, D = 4, 32, 64, 64, 64
ROWS = B * C
N = H * W * D
CENTER_H = (N // 2) // (W * D)
NORM = 65 * 65 * 65
LOG2E = 1.4426950408889634


def _entropy_body(x_ref, y_ref, out_ref, acc_ref, cen_ref):
    i = pl.program_id(0)

    for ref, row in ((x_ref, i), (y_ref, i + ROWS // 2)):
        blk = ref[0, 0]
        s = blk.sum(axis=0).sum(axis=0, keepdims=True)
        acc_ref[pl.ds(row, 1), :] = s
        cen_ref[pl.ds(row, 1), :] = blk[CENTER_H, 0:1, 0:1]

    @pl.when(i == ROWS // 2 - 1)
    def _():
        sums = acc_ref[...].sum(axis=1, keepdims=True)
        cen = cen_ref[...]
        nb = (sums - cen) * (1.0 / (N - 1))
        vals = jnp.concatenate([cen, nb], axis=0)
        idx = jnp.floor(vals * NBINS).astype(jnp.int32)
        valid = (vals >= 0.0) & (vals <= 1.0)
        idx = jnp.minimum(idx, NBINS - 1)
        bins = lax.broadcasted_iota(jnp.int32, (2 * ROWS, NBINS), 1)
        match = (idx == bins) & valid
        counts = jnp.sum(match.astype(jnp.float32), axis=0, keepdims=True)
        p = counts * (1.0 / NORM)
        e = -jnp.sum(p * (jnp.log(p + 1e-10) * LOG2E), axis=1, keepdims=True)
        out_ref[...] = e


def kernel(F):
    out = pl.pallas_call(
        _entropy_body,
        grid=(ROWS // 2,),
        in_specs=[
            pl.BlockSpec((1, 1, H, W, D), lambda i: (i // C, i % C, 0, 0, 0)),
            pl.BlockSpec(
                (1, 1, H, W, D),
                lambda i: ((i + ROWS // 2) // C, (i + ROWS // 2) % C, 0, 0, 0),
            ),
        ],
        out_specs=pl.BlockSpec((1, 1), lambda i: (0, 0)),
        out_shape=jax.ShapeDtypeStruct((1, 1), jnp.float32),
        scratch_shapes=[
            pltpu.VMEM((ROWS, D), jnp.float32),
            pltpu.VMEM((ROWS, 1), jnp.float32),
        ],
    )(F, F)
    return out.reshape(())


# speedup vs baseline: 4.2938x; 1.1089x over previous
import jax
import jax.numpy as jnp
from jax import lax
from jax.experimental import pallas as pl
from jax.experimental.pallas import tpu as pltpu

NBINS = 256
B, C, H, W, D = 4, 32, 64, 64, 64
ROWS = B * C
N = H * W * D
CENTER_H = (N // 2) // (W * D)
NORM = 65 * 65 * 65
LOG2E = 1.4426950408889634


NSTREAM = 4
SHARE = ROWS // NSTREAM


def _entropy_body(*refs):
    (in_refs, out_ref, acc_ref, cen_ref) = (
        refs[:NSTREAM], refs[NSTREAM], refs[NSTREAM + 1], refs[NSTREAM + 2])
    i = pl.program_id(0)

    for k, ref in enumerate(in_refs):
        row = i + k * SHARE
        blk = ref[0, 0]
        s = blk.sum(axis=0).sum(axis=0, keepdims=True)
        acc_ref[pl.ds(row, 1), :] = s
        cen_ref[pl.ds(row, 1), :] = blk[CENTER_H, 0:1, 0:1]

    @pl.when(i == SHARE - 1)
    def _():
        sums = acc_ref[...].sum(axis=1, keepdims=True)
        cen = cen_ref[...]
        nb = (sums - cen) * (1.0 / (N - 1))
        vals = jnp.concatenate([cen, nb], axis=0)
        idx = jnp.floor(vals * NBINS).astype(jnp.int32)
        valid = (vals >= 0.0) & (vals <= 1.0)
        idx = jnp.minimum(idx, NBINS - 1)
        bins = lax.broadcasted_iota(jnp.int32, (2 * ROWS, NBINS), 1)
        match = (idx == bins) & valid
        counts = jnp.sum(match.astype(jnp.float32), axis=0, keepdims=True)
        p = counts * (1.0 / NORM)
        e = -jnp.sum(p * (jnp.log(p + 1e-10) * LOG2E), axis=1, keepdims=True)
        out_ref[...] = e


def kernel(F):
    out = pl.pallas_call(
        _entropy_body,
        grid=(SHARE,),
        in_specs=[
            pl.BlockSpec(
                (1, 1, H, W, D),
                (lambda i, _k=k: ((i + _k * SHARE) // C, (i + _k * SHARE) % C,
                                  0, 0, 0)),
            )
            for k in range(NSTREAM)
        ],
        out_specs=pl.BlockSpec((1, 1), lambda i: (0, 0)),
        out_shape=jax.ShapeDtypeStruct((1, 1), jnp.float32),
        scratch_shapes=[
            pltpu.VMEM((ROWS, D), jnp.float32),
            pltpu.VMEM((ROWS, 1), jnp.float32),
        ],
    )(*([F] * NSTREAM))
    return out.reshape(())
